# baseline (speedup 1.0000x reference)
import functools

import jax
import jax.numpy as jnp
from jax import lax
from jax.experimental import pallas as pl
from jax.experimental.pallas import tpu as pltpu
from jax.experimental.pallas import tpu_sc as plsc

N_EMBD = 1024
VOCAB = 5
NPAIR = VOCAB * VOCAB

ROWS = 4 * 8192
NW = 32
RPW = ROWS // NW
CB = 32
NCH = RPW // CB
LANES = 16


def _sc_body(d_hbm, p_hbm, x_hbm, diao_hbm, out_hbm, dt, pt, xv, dv, idxv,
             bufs, comb_v, wsem0, wsem1):
    sid = lax.axis_index("s")
    wid = sid * 2 + lax.axis_index("c")
    base = wid * RPW

    pltpu.sync_copy(d_hbm, dt)
    pltpu.sync_copy(p_hbm, pt)
    pltpu.sync_copy(x_hbm.at[pl.ds(base, RPW)], xv)
    pltpu.sync_copy(diao_hbm.at[pl.ds(base, RPW)], dv)

    @pl.loop(0, N_EMBD // LANES)
    def build_comb(k):
        s = pl.ds(k * LANES, LANES)
        for i in range(VOCAB):
            drow = dt[i, s]
            for j in range(VOCAB):
                comb_v[i * VOCAB + j, s] = drow + pt[j, s]

    for k in range(RPW // LANES):
        s = pl.ds(k * LANES, LANES)
        idxv[s] = dv[s] * VOCAB + xv[s]

    wsems = (wsem0, wsem1)

    @pl.loop(0, NCH, step=2)
    def chunk_pair(i):
        for b in range(2):
            c = i + b

            @pl.when(i > 0)
            def _wait_prev():
                pltpu.make_async_copy(
                    bufs.at[b], out_hbm.at[pl.ds(base, CB)], wsems[b]
                ).wait()

            @plsc.parallel_loop(0, CB, unroll=2)
            def fill_row(r):
                iv = idxv[pl.ds(c * CB + r, LANES)]
                row = iv[0]
                for k in range(N_EMBD // LANES):
                    s = pl.ds(k * LANES, LANES)
                    bufs[b, r, s] = comb_v[row, s]

            pltpu.async_copy(
                bufs.at[b], out_hbm.at[pl.ds(base + c * CB, CB)], wsems[b]
            )

    for b in range(2):
        pltpu.make_async_copy(
            bufs.at[b], out_hbm.at[pl.ds(base, CB)], wsems[b]
        ).wait()


_sc_lookup = functools.partial(
    pl.kernel,
    out_type=jax.ShapeDtypeStruct((ROWS, N_EMBD), jnp.float32),
    mesh=plsc.VectorSubcoreMesh(core_axis_name="c", subcore_axis_name="s"),
    scratch_types=[
        pltpu.VMEM((VOCAB, N_EMBD), jnp.float32),
        pltpu.VMEM((VOCAB, N_EMBD), jnp.float32),
        pltpu.VMEM((RPW,), jnp.int32),
        pltpu.VMEM((RPW,), jnp.int32),
        pltpu.VMEM((RPW + LANES,), jnp.int32),
        pltpu.VMEM((2, CB, N_EMBD), jnp.float32),
        pltpu.VMEM((NPAIR, N_EMBD), jnp.float32),
        pltpu.SemaphoreType.DMA,
        pltpu.SemaphoreType.DMA,
    ],
)(_sc_body)


@jax.jit
def kernel(x, diao, diaoemb_weight, phemb_weight):
    xf = x.reshape(ROWS).astype(jnp.int32)
    df = diao.reshape(ROWS).astype(jnp.int32)
    out = _sc_lookup(diaoemb_weight, phemb_weight, xf, df)
    return out.reshape(x.shape[0], x.shape[1], N_EMBD)
